# Initial kernel scaffold; baseline (speedup 1.0000x reference)
import jax
import jax.numpy as jnp
from jax.experimental import pallas as pl


def kernel(x, incidence, edge_weights, W, b):
    raise NotImplementedError("write your pallas kernel here")


# speedup vs baseline: 5.7259x; 5.7259x over previous
import functools

import jax
import jax.numpy as jnp
from jax import lax
from jax.experimental import pallas as pl
from jax.experimental.pallas import tpu as pltpu
from jax.experimental.pallas import tpu_sc as plsc

N_NODES = 10000
N_HE = 10000
NNZ = 320000
D = 128
DH = 64
NPAD = 10240
NTILES = 16
ROWS = 2560
RPT = ROWS // NTILES
K = 4
ITERS = RPT // K
NR = NPAD // NTILES
SCH = 128
DUMMY = 10000


def _mm_body(x_ref, wt_ref, b_ref, lo_ref, hi_ref):
    s = jnp.dot(x_ref[...], wt_ref[...], preferred_element_type=jnp.float32)
    s = s + b_ref[...]
    lo_ref[...] = s[:, :DH]
    hi_ref[...] = s[:, DH:]


def _matmul(xp, wt, b2):
    blk = 1024
    return pl.pallas_call(
        _mm_body,
        grid=(NPAD // blk,),
        in_specs=[
            pl.BlockSpec((blk, D), lambda i: (i, 0)),
            pl.BlockSpec((D, D), lambda i: (0, 0)),
            pl.BlockSpec((1, D), lambda i: (0, 0)),
        ],
        out_specs=[
            pl.BlockSpec((blk, DH), lambda i: (i, 0)),
            pl.BlockSpec((blk, DH), lambda i: (i, 0)),
        ],
        out_shape=[jax.ShapeDtypeStruct((NPAD, DH), jnp.float32)] * 2,
    )(xp, wt, b2)


_MESH = plsc.VectorSubcoreMesh(core_axis_name="c", subcore_axis_name="s")


@functools.partial(
    pl.kernel,
    out_type=[jax.ShapeDtypeStruct((NPAD, DH), jnp.float32)] * 2,
    mesh=_MESH,
    scratch_types=[
        pltpu.VMEM_SHARED((NPAD, DH), jnp.float32),
        pltpu.VMEM_SHARED((NPAD, DH), jnp.float32),
        pltpu.VMEM((K, 128), jnp.int32),
        pltpu.VMEM((K, 128), jnp.int32),
        pltpu.VMEM((K * 128, DH), jnp.float32),
        pltpu.VMEM((SCH, 16), jnp.float32),
        pltpu.SemaphoreType.DMA,
    ],
    compiler_params=pltpu.CompilerParams(use_tc_tiling_on_sc=False),
)
def _sc_mp(suplo, suphi, nid2d, eid2d, wvec, outlo, outhi,
           he_acc, out_acc, idxn, idxe, rows, wbuf, sem):
    c = lax.axis_index("c")
    s = lax.axis_index("s")
    r0 = s * NR

    z = jnp.zeros((16,), jnp.float32)
    for r in range(32):
        for cc in range(DH // 16):
            rows[r, pl.ds(cc * 16, 16)] = z

    def zero_acc(t, carry):
        pltpu.sync_copy(rows.at[pl.ds(0, 32)], he_acc.at[pl.ds(r0 + t * 32, 32)])
        pltpu.sync_copy(rows.at[pl.ds(0, 32)], out_acc.at[pl.ds(r0 + t * 32, 32)])
        return carry

    lax.fori_loop(0, NR // 32, zero_acc, 0)
    plsc.subcore_barrier()

    def phase1(sup_ref):
        def outer(g, carry):
            rb = s * RPT + g * K
            pltpu.sync_copy(nid2d.at[pl.ds(rb, K)], idxn)
            pltpu.sync_copy(eid2d.at[pl.ds(rb, K)], idxe)
            cps = [
                pltpu.async_copy(
                    sup_ref.at[idxn.at[j]], rows.at[pl.ds(j * 128, 128)], sem)
                for j in range(K)
            ]
            for cp in cps:
                cp.wait()
            for j in range(K):
                pltpu.sync_copy(
                    rows.at[pl.ds(j * 128, 128)], he_acc.at[idxe.at[j]],
                    add=True)
            return carry

        lax.fori_loop(0, ITERS, outer, 0)

    @pl.when(c == 0)
    def _():
        phase1(suplo)

    @pl.when(c == 1)
    def _():
        phase1(suphi)

    plsc.subcore_barrier()

    def scale_chunk(ch, carry):
        rbase = r0 + ch * SCH
        pltpu.sync_copy(he_acc.at[pl.ds(rbase, SCH)], rows.at[pl.ds(0, SCH)])
        pltpu.sync_copy(wvec.at[pl.ds(rbase, SCH)], wbuf)

        def scale_grp(g, c2):
            for k in range(16):
                r = g * 16 + k
                wv = wbuf[r, pl.ds(0, 16)]
                for cc in range(DH // 16):
                    v = rows[r, pl.ds(cc * 16, 16)]
                    rows[r, pl.ds(cc * 16, 16)] = v * wv
            return c2

        lax.fori_loop(0, SCH // 16, scale_grp, 0)
        pltpu.sync_copy(rows.at[pl.ds(0, SCH)], he_acc.at[pl.ds(rbase, SCH)])
        return carry

    lax.fori_loop(0, NR // SCH, scale_chunk, 0)
    plsc.subcore_barrier()

    def outer2(g, carry):
        rb = s * RPT + g * K
        pltpu.sync_copy(nid2d.at[pl.ds(rb, K)], idxn)
        pltpu.sync_copy(eid2d.at[pl.ds(rb, K)], idxe)
        cps = [
            pltpu.async_copy(
                he_acc.at[idxe.at[j]], rows.at[pl.ds(j * 128, 128)], sem)
            for j in range(K)
        ]
        for cp in cps:
            cp.wait()
        for j in range(K):
            pltpu.sync_copy(
                rows.at[pl.ds(j * 128, 128)], out_acc.at[idxn.at[j]], add=True)
        return carry

    lax.fori_loop(0, ITERS, outer2, 0)
    plsc.subcore_barrier()

    @pl.when(c == 0)
    def _():
        pltpu.sync_copy(out_acc.at[pl.ds(r0, NR)], outlo.at[pl.ds(r0, NR)])

    @pl.when(c == 1)
    def _():
        pltpu.sync_copy(out_acc.at[pl.ds(r0, NR)], outhi.at[pl.ds(r0, NR)])


def kernel(x, incidence, edge_weights, W, b):
    node = incidence[0].astype(jnp.int32)
    he = incidence[1].astype(jnp.int32)
    npad = ROWS * 128 - NNZ
    nid2d = jnp.concatenate(
        [node, jnp.full((npad,), DUMMY, jnp.int32)]).reshape(ROWS, 128)
    eid2d = jnp.concatenate(
        [he, jnp.full((npad,), DUMMY, jnp.int32)]).reshape(ROWS, 128)
    wpad = jnp.concatenate(
        [edge_weights.astype(jnp.float32), jnp.zeros((NPAD - N_HE,), jnp.float32)])
    wrep = jnp.broadcast_to(wpad[:, None], (NPAD, 16))
    xp = jnp.concatenate(
        [x.astype(jnp.float32), jnp.zeros((NPAD - N_NODES, D), jnp.float32)])
    suplo, suphi = _matmul(xp, W.T.astype(jnp.float32),
                           b.astype(jnp.float32).reshape(1, D))
    lo, hi = _sc_mp(suplo, suphi, nid2d, eid2d, wrep)
    return jnp.concatenate([lo[:N_NODES], hi[:N_NODES]], axis=1)


# speedup vs baseline: 6.0640x; 1.0590x over previous
import functools

import jax
import jax.numpy as jnp
from jax import lax
from jax.experimental import pallas as pl
from jax.experimental.pallas import tpu as pltpu
from jax.experimental.pallas import tpu_sc as plsc

N_NODES = 10000
N_HE = 10000
NNZ = 320000
D = 128
DH = 64
NPAD = 10240
NTILES = 16
ROWS = 2560
RPT = ROWS // NTILES
K = 2
ITERS = RPT // K
NR = NPAD // NTILES
SCH = 128
DUMMY = 10000


def _mm_body(x_ref, wt_ref, b_ref, lo_ref, hi_ref):
    s = jnp.dot(x_ref[...], wt_ref[...], preferred_element_type=jnp.float32)
    s = s + b_ref[...]
    lo_ref[...] = s[:, :DH]
    hi_ref[...] = s[:, DH:]


def _matmul(xp, wt, b2):
    blk = 1024
    return pl.pallas_call(
        _mm_body,
        grid=(NPAD // blk,),
        in_specs=[
            pl.BlockSpec((blk, D), lambda i: (i, 0)),
            pl.BlockSpec((D, D), lambda i: (0, 0)),
            pl.BlockSpec((1, D), lambda i: (0, 0)),
        ],
        out_specs=[
            pl.BlockSpec((blk, DH), lambda i: (i, 0)),
            pl.BlockSpec((blk, DH), lambda i: (i, 0)),
        ],
        out_shape=[jax.ShapeDtypeStruct((NPAD, DH), jnp.float32)] * 2,
    )(xp, wt, b2)


_MESH = plsc.VectorSubcoreMesh(core_axis_name="c", subcore_axis_name="s")


@functools.partial(
    pl.kernel,
    out_type=[jax.ShapeDtypeStruct((NPAD, DH), jnp.float32)] * 2,
    mesh=_MESH,
    scratch_types=[
        pltpu.VMEM_SHARED((NPAD, DH), jnp.float32),
        pltpu.VMEM_SHARED((NPAD, DH), jnp.float32),
        pltpu.VMEM((2, K, 2, 128), jnp.int32),
        pltpu.VMEM((2, K * 128, DH), jnp.float32),
        pltpu.VMEM((SCH, 16), jnp.float32),
        pltpu.SemaphoreType.DMA,
        pltpu.SemaphoreType.DMA,
    ],
    compiler_params=pltpu.CompilerParams(use_tc_tiling_on_sc=False),
)
def _sc_mp(suplo, suphi, ids3d, wvec, outlo, outhi,
           he_acc, out_acc, idxb, rows, wbuf, semA, semB):
    c = lax.axis_index("c")
    s = lax.axis_index("s")
    r0 = s * NR

    z = jnp.zeros((16,), jnp.float32)
    for r in range(32):
        for cc in range(DH // 16):
            rows[0, r, pl.ds(cc * 16, 16)] = z

    def zero_acc(t, carry):
        pltpu.sync_copy(rows.at[0, pl.ds(0, 32)],
                        he_acc.at[pl.ds(r0 + t * 32, 32)])
        pltpu.sync_copy(rows.at[0, pl.ds(0, 32)],
                        out_acc.at[pl.ds(r0 + t * 32, 32)])
        return carry

    lax.fori_loop(0, NR // 32, zero_acc, 0)
    plsc.subcore_barrier()

    def mp_phase(src_table, acc, gwhich, swhich):
        sems = (semA, semB)

        def load_fire(g, sl):
            rb = s * RPT + g * K
            pltpu.sync_copy(ids3d.at[pl.ds(rb, K)], idxb.at[sl])
            for j in range(K):
                pltpu.async_copy(
                    src_table.at[idxb.at[sl, j, gwhich]],
                    rows.at[sl, pl.ds(j * 128, 128)], sems[sl])

        def drain_scatter(g, sl):
            for j in range(K):
                pltpu.make_async_copy(
                    src_table.at[idxb.at[sl, j, gwhich]],
                    rows.at[sl, pl.ds(j * 128, 128)], sems[sl]).wait()
            for j in range(K):
                pltpu.sync_copy(
                    rows.at[sl, pl.ds(j * 128, 128)],
                    acc.at[idxb.at[sl, j, swhich]], add=True)

        load_fire(0, 0)

        def body(i, carry):
            g = 2 * i
            load_fire(g + 1, 1)
            drain_scatter(g, 0)
            load_fire(g + 2, 0)
            drain_scatter(g + 1, 1)
            return carry

        lax.fori_loop(0, ITERS // 2 - 1, body, 0)
        g = ITERS - 2
        load_fire(g + 1, 1)
        drain_scatter(g, 0)
        drain_scatter(g + 1, 1)

    @pl.when(c == 0)
    def _():
        mp_phase(suplo, he_acc, 0, 1)

    @pl.when(c == 1)
    def _():
        mp_phase(suphi, he_acc, 0, 1)

    plsc.subcore_barrier()

    def scale_chunk(ch, carry):
        rbase = r0 + ch * SCH
        pltpu.sync_copy(he_acc.at[pl.ds(rbase, SCH)], rows.at[0, pl.ds(0, SCH)])
        pltpu.sync_copy(wvec.at[pl.ds(rbase, SCH)], wbuf)

        def scale_grp(g, c2):
            for k in range(16):
                r = g * 16 + k
                wv = wbuf[r, pl.ds(0, 16)]
                for cc in range(DH // 16):
                    v = rows[0, r, pl.ds(cc * 16, 16)]
                    rows[0, r, pl.ds(cc * 16, 16)] = v * wv
            return c2

        lax.fori_loop(0, SCH // 16, scale_grp, 0)
        pltpu.sync_copy(rows.at[0, pl.ds(0, SCH)], he_acc.at[pl.ds(rbase, SCH)])
        return carry

    lax.fori_loop(0, NR // SCH, scale_chunk, 0)
    plsc.subcore_barrier()

    mp_phase(he_acc, out_acc, 1, 0)
    plsc.subcore_barrier()

    @pl.when(c == 0)
    def _():
        pltpu.sync_copy(out_acc.at[pl.ds(r0, NR)], outlo.at[pl.ds(r0, NR)])

    @pl.when(c == 1)
    def _():
        pltpu.sync_copy(out_acc.at[pl.ds(r0, NR)], outhi.at[pl.ds(r0, NR)])


def kernel(x, incidence, edge_weights, W, b):
    node = incidence[0].astype(jnp.int32)
    he = incidence[1].astype(jnp.int32)
    npad = ROWS * 128 - NNZ
    nid2d = jnp.concatenate(
        [node, jnp.full((npad,), DUMMY, jnp.int32)]).reshape(ROWS, 128)
    eid2d = jnp.concatenate(
        [he, jnp.full((npad,), DUMMY, jnp.int32)]).reshape(ROWS, 128)
    ids3d = jnp.stack([nid2d, eid2d], axis=1)
    wpad = jnp.concatenate(
        [edge_weights.astype(jnp.float32), jnp.zeros((NPAD - N_HE,), jnp.float32)])
    wrep = jnp.broadcast_to(wpad[:, None], (NPAD, 16))
    xp = jnp.concatenate(
        [x.astype(jnp.float32), jnp.zeros((NPAD - N_NODES, D), jnp.float32)])
    suplo, suphi = _matmul(xp, W.T.astype(jnp.float32),
                           b.astype(jnp.float32).reshape(1, D))
    lo, hi = _sc_mp(suplo, suphi, ids3d, wrep)
    return jnp.concatenate([lo[:N_NODES], hi[:N_NODES]], axis=1)


# speedup vs baseline: 7.0155x; 1.1569x over previous
import functools

import jax
import jax.numpy as jnp
from jax import lax
from jax.experimental import pallas as pl
from jax.experimental.pallas import tpu as pltpu
from jax.experimental.pallas import tpu_sc as plsc

N_NODES = 10000
N_HE = 10000
NNZ = 320000
D = 128
DH = 64
NPAD = 10240
NTILES = 16
ROWS = 2560
RPT = ROWS // NTILES
K = 2
ITERS = RPT // K
NR = NPAD // NTILES
SCH = 128
DUMMY = 10000


def _mm_body(x_ref, wt_ref, b_ref, lo_ref, hi_ref):
    s = jnp.dot(x_ref[...], wt_ref[...], preferred_element_type=jnp.float32)
    s = s + b_ref[...]
    lo_ref[...] = s[:, :DH]
    hi_ref[...] = s[:, DH:]


def _matmul(xp, wt, b2):
    blk = 1024
    return pl.pallas_call(
        _mm_body,
        grid=(NPAD // blk,),
        in_specs=[
            pl.BlockSpec((blk, D), lambda i: (i, 0)),
            pl.BlockSpec((D, D), lambda i: (0, 0)),
            pl.BlockSpec((1, D), lambda i: (0, 0)),
        ],
        out_specs=[
            pl.BlockSpec((blk, DH), lambda i: (i, 0)),
            pl.BlockSpec((blk, DH), lambda i: (i, 0)),
        ],
        out_shape=[jax.ShapeDtypeStruct((NPAD, DH), jnp.float32)] * 2,
    )(xp, wt, b2)


_MESH = plsc.VectorSubcoreMesh(core_axis_name="c", subcore_axis_name="s")


@functools.partial(
    pl.kernel,
    out_type=[jax.ShapeDtypeStruct((NPAD, DH), jnp.float32)] * 2,
    mesh=_MESH,
    scratch_types=[
        pltpu.VMEM_SHARED((NPAD, DH), jnp.float32),
        pltpu.VMEM_SHARED((NPAD, DH), jnp.float32),
        pltpu.VMEM((4, K, 2, 128), jnp.int32),
        pltpu.VMEM((2, K * 128, DH), jnp.float32),
        pltpu.VMEM((SCH, 16), jnp.float32),
        pltpu.SemaphoreType.DMA,
        pltpu.SemaphoreType.DMA,
        pltpu.SemaphoreType.DMA,
        pltpu.SemaphoreType.DMA,
        pltpu.SemaphoreType.DMA,
        pltpu.SemaphoreType.DMA,
        pltpu.SemaphoreType.DMA,
        pltpu.SemaphoreType.DMA,
    ],
    compiler_params=pltpu.CompilerParams(use_tc_tiling_on_sc=False),
)
def _sc_mp(suplo, suphi, ids3d, wvec, outlo, outhi,
           he_acc, out_acc, idxb, rows, wbuf,
           semG0, semG1, semS0, semS1, semI0, semI1, semI2, semI3):
    c = lax.axis_index("c")
    s = lax.axis_index("s")
    r0 = s * NR

    z = jnp.zeros((16,), jnp.float32)
    for r in range(32):
        for cc in range(DH // 16):
            rows[0, r, pl.ds(cc * 16, 16)] = z

    def zero_acc(t, carry):
        pltpu.sync_copy(rows.at[0, pl.ds(0, 32)],
                        he_acc.at[pl.ds(r0 + t * 32, 32)])
        pltpu.sync_copy(rows.at[0, pl.ds(0, 32)],
                        out_acc.at[pl.ds(r0 + t * 32, 32)])
        return carry

    lax.fori_loop(0, NR // 32, zero_acc, 0)
    plsc.subcore_barrier()

    def mp_phase(src_table, acc, gwhich, swhich):
        semG = (semG0, semG1)
        semS = (semS0, semS1)
        semI = (semI0, semI1, semI2, semI3)

        def idx_load(g, isl):
            rb = s * RPT + g * K
            pltpu.async_copy(ids3d.at[pl.ds(rb, K)], idxb.at[isl], semI[isl])

        def wait_idx(isl):
            pltpu.make_async_copy(
                ids3d.at[pl.ds(0, K)], idxb.at[isl], semI[isl]).wait()

        def fire_g(g, rs, isl):
            for j in range(K):
                pltpu.async_copy(
                    src_table.at[idxb.at[isl, j, gwhich]],
                    rows.at[rs, pl.ds(j * 128, 128)], semG[rs])

        def wait_g(rs, isl):
            for j in range(K):
                pltpu.make_async_copy(
                    src_table.at[idxb.at[isl, j, gwhich]],
                    rows.at[rs, pl.ds(j * 128, 128)], semG[rs]).wait()

        def fire_s(rs, isl):
            for j in range(K):
                pltpu.async_copy(
                    rows.at[rs, pl.ds(j * 128, 128)],
                    acc.at[idxb.at[isl, j, swhich]], semS[rs], add=True)

        def wait_s(rs, isl):
            for j in range(K):
                pltpu.make_async_copy(
                    rows.at[rs, pl.ds(j * 128, 128)],
                    acc.at[idxb.at[isl, j, swhich]], semS[rs]).wait()

        def stage(g, rs, isl, load):
            wait_s(rs, (isl + 2) % 4)
            if load:
                idx_load(g + 2, (isl + 2) % 4)
            wait_idx(isl)
            fire_g(g, rs, isl)
            wait_g(1 - rs, (isl + 3) % 4)
            fire_s(1 - rs, (isl + 3) % 4)

        idx_load(0, 0)
        idx_load(1, 1)
        idx_load(2, 2)
        wait_idx(0)
        fire_g(0, 0, 0)
        idx_load(3, 3)
        wait_idx(1)
        fire_g(1, 1, 1)
        wait_g(0, 0)
        fire_s(0, 0)

        def body(i, carry):
            g = 4 * i + 2
            stage(g, 0, 2, True)
            stage(g + 1, 1, 3, True)
            stage(g + 2, 0, 0, True)
            stage(g + 3, 1, 1, True)
            return carry

        lax.fori_loop(0, (ITERS - 4) // 4, body, 0)
        g = ITERS - 2
        stage(g, 0, 2, False)
        stage(g + 1, 1, 3, False)
        wait_g(1, 3)
        fire_s(1, 3)
        wait_s(0, 2)
        wait_s(1, 3)

    @pl.when(c == 0)
    def _():
        mp_phase(suplo, he_acc, 0, 1)

    @pl.when(c == 1)
    def _():
        mp_phase(suphi, he_acc, 0, 1)

    plsc.subcore_barrier()

    def scale_chunk(ch, carry):
        rbase = r0 + ch * SCH
        pltpu.sync_copy(he_acc.at[pl.ds(rbase, SCH)], rows.at[0, pl.ds(0, SCH)])
        pltpu.sync_copy(wvec.at[pl.ds(rbase, SCH)], wbuf)

        def scale_grp(g, c2):
            for k in range(16):
                r = g * 16 + k
                wv = wbuf[r, pl.ds(0, 16)]
                for cc in range(DH // 16):
                    v = rows[0, r, pl.ds(cc * 16, 16)]
                    rows[0, r, pl.ds(cc * 16, 16)] = v * wv
            return c2

        lax.fori_loop(0, SCH // 16, scale_grp, 0)
        pltpu.sync_copy(rows.at[0, pl.ds(0, SCH)], he_acc.at[pl.ds(rbase, SCH)])
        return carry

    lax.fori_loop(0, NR // SCH, scale_chunk, 0)
    plsc.subcore_barrier()

    mp_phase(he_acc, out_acc, 1, 0)
    plsc.subcore_barrier()

    @pl.when(c == 0)
    def _():
        pltpu.sync_copy(out_acc.at[pl.ds(r0, NR)], outlo.at[pl.ds(r0, NR)])

    @pl.when(c == 1)
    def _():
        pltpu.sync_copy(out_acc.at[pl.ds(r0, NR)], outhi.at[pl.ds(r0, NR)])


def kernel(x, incidence, edge_weights, W, b):
    node = incidence[0].astype(jnp.int32)
    he = incidence[1].astype(jnp.int32)
    npad = ROWS * 128 - NNZ
    nid2d = jnp.concatenate(
        [node, jnp.full((npad,), DUMMY, jnp.int32)]).reshape(ROWS, 128)
    eid2d = jnp.concatenate(
        [he, jnp.full((npad,), DUMMY, jnp.int32)]).reshape(ROWS, 128)
    ids3d = jnp.stack([nid2d, eid2d], axis=1)
    wpad = jnp.concatenate(
        [edge_weights.astype(jnp.float32), jnp.zeros((NPAD - N_HE,), jnp.float32)])
    wrep = jnp.broadcast_to(wpad[:, None], (NPAD, 16))
    xp = jnp.concatenate(
        [x.astype(jnp.float32), jnp.zeros((NPAD - N_NODES, D), jnp.float32)])
    suplo, suphi = _matmul(xp, W.T.astype(jnp.float32),
                           b.astype(jnp.float32).reshape(1, D))
    lo, hi = _sc_mp(suplo, suphi, ids3d, wrep)
    return jnp.concatenate([lo[:N_NODES], hi[:N_NODES]], axis=1)
